# baseline (speedup 1.0000x reference)
import jax
import jax.numpy as jnp
from jax.experimental import pallas as pl
from jax.experimental.pallas import tpu as pltpu

IN_DIMS = 1024
SEQ_LEN = 2048
TOP_K = 64
BLK = 256
N_BISECT = 13


def _fused_body(xf_ref, w_ref, b_ref, out_ref, xn_ref, colsum_ref):
    i = pl.program_id(1)

    @pl.when(i == 0)
    def _init():
        xf32 = xf_ref[0].astype(jnp.float32)
        rn = jax.lax.rsqrt(jnp.maximum(
            jnp.sum(xf32 * xf32, axis=1, keepdims=True), 1e-24))
        xn_ref[...] = (xf32 * rn).astype(jnp.bfloat16)
        colsum_ref[...] = jnp.sum(xf32, axis=0, keepdims=True)

    xn = xn_ref[...]
    xnb = xn_ref[pl.ds(i * BLK, BLK), :]
    scores = jax.lax.dot_general(
        xnb, xn, (((1,), (1,)), ((), ())),
        preferred_element_type=jnp.float32).astype(jnp.bfloat16)

    lo0 = jnp.full((BLK, 1), -1.01, jnp.float32)
    hi0 = jnp.full((BLK, 1), 1.01, jnp.float32)
    cnt0 = jnp.full((BLK, 1), float(SEQ_LEN), jnp.float32)

    def body(_, carry):
        lo, hi, cnt_lo = carry
        mid = 0.5 * (lo + hi)
        mask = scores >= mid.astype(jnp.bfloat16)
        cnt = jnp.sum(mask.astype(jnp.bfloat16), axis=1,
                      keepdims=True).astype(jnp.float32)
        ge = cnt >= TOP_K
        return (jnp.where(ge, mid, lo), jnp.where(ge, hi, mid),
                jnp.where(ge, cnt, cnt_lo))

    lo, _, cnt = jax.lax.fori_loop(0, N_BISECT, body, (lo0, hi0, cnt0))

    m = scores >= lo.astype(jnp.bfloat16)
    e = jnp.exp(scores.astype(jnp.float32))
    w = jnp.where(m, e - 1.0, 0.0)
    sumexp = jnp.sum(jnp.where(m, e, 0.0), axis=1, keepdims=True)
    denom = sumexp + (SEQ_LEN - cnt)

    wx = jax.lax.dot_general(
        w.astype(jnp.bfloat16), xf_ref[0], (((1,), (0,)), ((), ())),
        preferred_element_type=jnp.float32)
    g = (wx + colsum_ref[...]) / denom
    out = jax.lax.dot_general(
        g.astype(jnp.bfloat16), w_ref[...], (((1,), (1,)), ((), ())),
        preferred_element_type=jnp.float32)
    out_ref[0] = out + b_ref[...]


def kernel(x, W, b):
    B, S, D = x.shape
    nblk = S // BLK
    b2 = b.reshape(1, D)
    xb16 = x.astype(jnp.bfloat16)
    Wb16 = W.astype(jnp.bfloat16)
    out = pl.pallas_call(
        _fused_body,
        grid=(B, nblk),
        in_specs=[
            pl.BlockSpec((1, S, D), lambda bi, i: (bi, 0, 0)),
            pl.BlockSpec((D, D), lambda bi, i: (0, 0)),
            pl.BlockSpec((1, D), lambda bi, i: (0, 0)),
        ],
        out_specs=pl.BlockSpec((1, BLK, D), lambda bi, i: (bi, i, 0)),
        out_shape=jax.ShapeDtypeStruct((B, S, D), jnp.float32),
        scratch_shapes=[
            pltpu.VMEM((S, D), jnp.bfloat16),
            pltpu.VMEM((1, D), jnp.float32),
        ],
    )(xb16, Wb16, b2)
    return out


# speedup vs baseline: 1.1995x; 1.1995x over previous
import jax
import jax.numpy as jnp
from jax.experimental import pallas as pl
from jax.experimental.pallas import tpu as pltpu

IN_DIMS = 1024
SEQ_LEN = 2048
TOP_K = 64
BLK = 256
N_BISECT = 13


def _fused_body(xf_ref, w_ref, b_ref, out_ref, xn_ref):
    i = pl.program_id(1)

    @pl.when(i == 0)
    def _init():
        xf32 = xf_ref[0].astype(jnp.float32)
        rn = jax.lax.rsqrt(jnp.maximum(
            jnp.sum(xf32 * xf32, axis=1, keepdims=True), 1e-24))
        xn_ref[...] = (xf32 * rn).astype(jnp.bfloat16)

    xn = xn_ref[...]
    xnb = xn_ref[pl.ds(i * BLK, BLK), :]
    scores = jax.lax.dot_general(
        xn, xnb, (((1,), (1,)), ((), ())),
        preferred_element_type=jnp.float32)

    lo0 = jnp.full((1, BLK), -1.01, jnp.float32)
    hi0 = jnp.full((1, BLK), 1.01, jnp.float32)

    def body(_, carry):
        lo, hi = carry
        mid = 0.5 * (lo + hi)
        cnt = jnp.sum((scores >= mid).astype(jnp.float32), axis=0,
                      keepdims=True)
        ge = cnt >= TOP_K
        return jnp.where(ge, mid, lo), jnp.where(ge, hi, mid)

    lo, _ = jax.lax.fori_loop(0, N_BISECT, body, (lo0, hi0))

    e = jnp.exp(scores)
    a = jnp.where(scores >= lo, e, 1.0)
    denom = jnp.sum(a, axis=0, keepdims=True)
    attn = (a * (1.0 / denom)).astype(jnp.bfloat16)

    g = jax.lax.dot_general(
        attn, xf_ref[0], (((0,), (0,)), ((), ())),
        preferred_element_type=jnp.float32)
    out = jax.lax.dot_general(
        g.astype(jnp.bfloat16), w_ref[...], (((1,), (1,)), ((), ())),
        preferred_element_type=jnp.float32)
    out_ref[0] = out + b_ref[...]


def kernel(x, W, b):
    B, S, D = x.shape
    nblk = S // BLK
    b2 = b.reshape(1, D)
    xb16 = x.astype(jnp.bfloat16)
    Wb16 = W.astype(jnp.bfloat16)
    out = pl.pallas_call(
        _fused_body,
        grid=(B, nblk),
        in_specs=[
            pl.BlockSpec((1, S, D), lambda bi, i: (bi, 0, 0)),
            pl.BlockSpec((D, D), lambda bi, i: (0, 0)),
            pl.BlockSpec((1, D), lambda bi, i: (0, 0)),
        ],
        out_specs=pl.BlockSpec((1, BLK, D), lambda bi, i: (bi, i, 0)),
        out_shape=jax.ShapeDtypeStruct((B, S, D), jnp.float32),
        scratch_shapes=[
            pltpu.VMEM((S, D), jnp.bfloat16),
        ],
    )(xb16, Wb16, b2)
    return out


# speedup vs baseline: 1.2661x; 1.0556x over previous
import jax
import jax.numpy as jnp
from jax.experimental import pallas as pl
from jax.experimental.pallas import tpu as pltpu

IN_DIMS = 1024
SEQ_LEN = 2048
TOP_K = 64
BLK = 256
N_BISECT = 10


def _fused_body(xf_ref, w_ref, b_ref, out_ref, xn_ref):
    i = pl.program_id(1)

    @pl.when(i == 0)
    def _init():
        xf32 = xf_ref[0].astype(jnp.float32)
        rn = jax.lax.rsqrt(jnp.maximum(
            jnp.sum(xf32 * xf32, axis=1, keepdims=True), 1e-24))
        xn_ref[...] = (xf32 * rn).astype(jnp.bfloat16)

    xn = xn_ref[...]
    xnb = xn_ref[pl.ds(i * BLK, BLK), :]
    scores = jax.lax.dot_general(
        xn, xnb, (((1,), (1,)), ((), ())),
        preferred_element_type=jnp.float32)

    lo0 = jnp.full((1, BLK), -1.01, jnp.float32)
    hi0 = jnp.full((1, BLK), 1.01, jnp.float32)

    lo, hi = lo0, hi0
    for _ in range(N_BISECT):
        mid = 0.5 * (lo + hi)
        cnt = jnp.sum((scores >= mid).astype(jnp.float32), axis=0,
                      keepdims=True)
        ge = cnt >= TOP_K
        lo, hi = jnp.where(ge, mid, lo), jnp.where(ge, hi, mid)

    e = jnp.exp(scores)
    a = jnp.where(scores >= lo, e, 1.0)
    denom = jnp.sum(a, axis=0, keepdims=True)
    attn = (a * (1.0 / denom)).astype(jnp.bfloat16)

    g = jax.lax.dot_general(
        attn, xf_ref[0], (((0,), (0,)), ((), ())),
        preferred_element_type=jnp.float32)
    out = jax.lax.dot_general(
        g.astype(jnp.bfloat16), w_ref[...], (((1,), (1,)), ((), ())),
        preferred_element_type=jnp.float32)
    out_ref[0] = out + b_ref[...]


def kernel(x, W, b):
    B, S, D = x.shape
    nblk = S // BLK
    b2 = b.reshape(1, D)
    xb16 = x.astype(jnp.bfloat16)
    Wb16 = W.astype(jnp.bfloat16)
    out = pl.pallas_call(
        _fused_body,
        grid=(B, nblk),
        in_specs=[
            pl.BlockSpec((1, S, D), lambda bi, i: (bi, 0, 0)),
            pl.BlockSpec((D, D), lambda bi, i: (0, 0)),
            pl.BlockSpec((1, D), lambda bi, i: (0, 0)),
        ],
        out_specs=pl.BlockSpec((1, BLK, D), lambda bi, i: (bi, i, 0)),
        out_shape=jax.ShapeDtypeStruct((B, S, D), jnp.float32),
        scratch_shapes=[
            pltpu.VMEM((S, D), jnp.bfloat16),
        ],
    )(xb16, Wb16, b2)
    return out
